# baseline (speedup 1.0000x reference)
import jax
import jax.numpy as jnp
import numpy as np
from jax import lax
from jax.experimental import pallas as pl
from jax.experimental.pallas import tpu as pltpu
from jax.experimental.pallas import tpu_sc as plsc

B, P, D = 4096, 2048, 32
M = B * P
FMIN = float(np.finfo(np.float32).min)
DN = (((1,), (0,)), ((), ()))


_NC = 2
_NS = 16
_NW = _NC * _NS
_ROWS_PER_W = B // _NW


def _sc_gather_body(x_hbm, fc_hbm, out_hbm, xrow, fcrow, orow):
    c = lax.axis_index("c")
    s = lax.axis_index("s")
    wid = s * _NC + c
    base = wid * _ROWS_PER_W

    def row_fn(r, _):
        row = base + r
        pltpu.sync_copy(x_hbm.at[row], xrow)
        pltpu.sync_copy(fc_hbm.at[row], fcrow)

        def col_fn(j, _):
            xv = xrow[pl.ds(j * 16, 16)]
            xi = xv.astype(jnp.int32)
            g = plsc.load_gather(fcrow, [xi])
            orow[pl.ds(j * 16, 16)] = g
            return 0

        lax.fori_loop(0, P // 16, col_fn, 0)
        pltpu.sync_copy(orow, out_hbm.at[row])
        return 0

    lax.fori_loop(0, _ROWS_PER_W, row_fn, 0)


def _sc_gather(xplane, fcplane):
    mesh = plsc.VectorSubcoreMesh(core_axis_name="c", subcore_axis_name="s")
    fn = pl.kernel(
        _sc_gather_body,
        mesh=mesh,
        out_type=jax.ShapeDtypeStruct((B, P), jnp.float32),
        scratch_types=[
            pltpu.VMEM((P,), jnp.float32),
            pltpu.VMEM((P,), jnp.float32),
            pltpu.VMEM((P,), jnp.float32),
        ],
        compiler_params=pltpu.CompilerParams(needs_layout_passes=False),
    )
    return fn(xplane, fcplane)


_PK = 8
_RT = 2048
_MP = M // _PK


def _head_body(s_ref, w1_ref, b1_ref, w2_ref, b2_ref, out_ref):
    s = s_ref[...]
    G = lax.dot_general(s, w1_ref[...], DN, preferred_element_type=jnp.float32)
    h = jnp.maximum(G + b1_ref[...], 0.0)
    l = lax.dot_general(h, w2_ref[...], DN, preferred_element_type=jnp.float32)
    out_ref[...] = l + b2_ref[...]


def _mlp_head(s5, W1, b1, W2, b2):
    s5p = s5.reshape(_MP, 5 * _PK)
    eye = jnp.eye(_PK, dtype=jnp.float32)
    w1p = jnp.kron(eye, W1)
    b1p = jnp.tile(b1.reshape(1, D), (1, _PK))
    w2p = jnp.kron(eye, W2)
    out = pl.pallas_call(
        _head_body,
        grid=(_MP // _RT,),
        in_specs=[
            pl.BlockSpec((_RT, 5 * _PK), lambda i: (i, 0)),
            pl.BlockSpec((5 * _PK, D * _PK), lambda i: (0, 0)),
            pl.BlockSpec((1, D * _PK), lambda i: (0, 0)),
            pl.BlockSpec((D * _PK, _PK), lambda i: (0, 0)),
            pl.BlockSpec((1, 1), lambda i: (0, 0)),
        ],
        out_specs=pl.BlockSpec((_RT, _PK), lambda i: (i, 0)),
        out_shape=jax.ShapeDtypeStruct((_MP, _PK), jnp.float32),
        compiler_params=pltpu.CompilerParams(
            dimension_semantics=("arbitrary",),
        ),
    )(s5p, w1p, b1p, w2p, b2.reshape(1, 1))
    return out.reshape(M, 1)


_BT = 128


def _item_sel_body(l_ref, w_ref, item_ref, lp_ref, iw_ref):
    l = l_ref[...]
    m = jnp.max(l, axis=-1, keepdims=True)
    ex = jnp.exp(l - m)
    s = jnp.sum(ex, axis=-1, keepdims=True)
    probs = ex / s
    pm = jnp.max(probs, axis=-1, keepdims=True)
    iota = lax.broadcasted_iota(jnp.int32, l.shape, 1)
    item = jnp.min(jnp.where(probs == pm, iota, P), axis=-1, keepdims=True)
    item_ref[...] = item
    lp_ref[...] = jnp.log(pm)
    iw_ref[...] = jnp.sum(jnp.where(iota == item, w_ref[...], 0.0),
                          axis=-1, keepdims=True)


def _item_select(l_item, wplane):
    return pl.pallas_call(
        _item_sel_body,
        grid=(B // _BT,),
        in_specs=[
            pl.BlockSpec((_BT, P), lambda i: (i, 0)),
            pl.BlockSpec((_BT, P), lambda i: (i, 0)),
        ],
        out_specs=[
            pl.BlockSpec((_BT, 1), lambda i: (i, 0)),
            pl.BlockSpec((_BT, 1), lambda i: (i, 0)),
            pl.BlockSpec((_BT, 1), lambda i: (i, 0)),
        ],
        out_shape=[
            jax.ShapeDtypeStruct((B, 1), jnp.int32),
            jax.ShapeDtypeStruct((B, 1), jnp.float32),
            jax.ShapeDtypeStruct((B, 1), jnp.float32),
        ],
        compiler_params=pltpu.CompilerParams(
            dimension_semantics=("arbitrary",),
        ),
    )(l_item, wplane)


def _bin_sel_body(l_ref, fc_ref, item_ref, iw_ref, lpi_ref, act_ref, lp_ref):
    l = l_ref[...]
    fc = fc_ref[...]
    iw = iw_ref[...]
    item = item_ref[...]
    iota = lax.broadcasted_iota(jnp.int32, l.shape, 1)
    over = jnp.where(iw - fc > 0.0, FMIN, 0.0)
    mask = jnp.where(iota == item, FMIN, over)
    lm = l + mask
    m = jnp.max(lm, axis=-1, keepdims=True)
    ex = jnp.exp(lm - m)
    s = jnp.sum(ex, axis=-1, keepdims=True)
    probs = ex / s
    pm = jnp.max(probs, axis=-1, keepdims=True)
    bin_ = jnp.min(jnp.where(probs == pm, iota, P), axis=-1, keepdims=True)
    act_ref[...] = jnp.concatenate([item, bin_], axis=1)
    lp_ref[...] = lpi_ref[...] + jnp.log(pm)


def _bin_select(l_bin, fcplane, item, iw, lp_i):
    return pl.pallas_call(
        _bin_sel_body,
        grid=(B // _BT,),
        in_specs=[
            pl.BlockSpec((_BT, P), lambda i: (i, 0)),
            pl.BlockSpec((_BT, P), lambda i: (i, 0)),
            pl.BlockSpec((_BT, 1), lambda i: (i, 0)),
            pl.BlockSpec((_BT, 1), lambda i: (i, 0)),
            pl.BlockSpec((_BT, 1), lambda i: (i, 0)),
        ],
        out_specs=[
            pl.BlockSpec((_BT, 2), lambda i: (i, 0)),
            pl.BlockSpec((_BT, 1), lambda i: (i, 0)),
        ],
        out_shape=[
            jax.ShapeDtypeStruct((B, 2), jnp.int32),
            jax.ShapeDtypeStruct((B, 1), jnp.float32),
        ],
        compiler_params=pltpu.CompilerParams(
            dimension_semantics=("arbitrary",),
        ),
    )(l_bin, fcplane, item, iw, lp_i)


def kernel(state, W1i, b1i, W2i, b2i, W1b, b1b, W2b, b2b, greedy=1):
    wpl = state[..., 1]
    fcpl = state[..., 2]
    fci = jnp.take_along_axis(fcpl, state[..., 0].astype(jnp.int32), axis=1)
    s5i = jnp.stack([wpl, fci, state[..., 3], state[..., 4], state[..., 5]],
                    axis=-1).reshape(M, 5)
    l_item = _mlp_head(s5i, W1i, b1i, W2i, b2i).reshape(B, P)
    item, lp_i, iw = _item_select(l_item, wpl)
    s5b = jnp.stack([jnp.broadcast_to(iw, (B, P)), fcpl, state[..., 3],
                     state[..., 4], state[..., 5]], axis=-1).reshape(M, 5)
    l_bin = _mlp_head(s5b, W1b, b1b, W2b, b2b).reshape(B, P)
    action, lp = _bin_select(l_bin, fcpl, item, iw, lp_i)
    return action, lp


# speedup vs baseline: 1.0713x; 1.0713x over previous
import jax
import jax.numpy as jnp
import numpy as np
from jax import lax
from jax.experimental import pallas as pl
from jax.experimental.pallas import tpu as pltpu
from jax.experimental.pallas import tpu_sc as plsc

B, P, D = 4096, 2048, 32
M = B * P
FMIN = float(np.finfo(np.float32).min)
DN = (((1,), (0,)), ((), ()))


_NC = 2
_NS = 16
_NW = _NC * _NS
_ROWS_PER_W = B // _NW
_RW = P * 6


def _sc_gather_body(state_hbm, fci_hbm, w_hbm, fc_hbm,
                    rowbuf, orow, wrow, fcrow):
    c = lax.axis_index("c")
    s = lax.axis_index("s")
    wid = s * _NC + c
    base = wid * _ROWS_PER_W
    iota6 = lax.iota(jnp.int32, 16) * 6

    def row_fn(r, _):
        row = base + r
        pltpu.sync_copy(state_hbm.at[row], rowbuf)

        def col_fn(j, _):
            idx = iota6 + j * 96
            xv = plsc.load_gather(rowbuf, [idx])
            wv = plsc.load_gather(rowbuf, [idx + 1])
            fcv = plsc.load_gather(rowbuf, [idx + 2])
            xi = xv.astype(jnp.int32) * 6 + 2
            g = plsc.load_gather(rowbuf, [xi])
            orow[pl.ds(j * 16, 16)] = g
            wrow[pl.ds(j * 16, 16)] = wv
            fcrow[pl.ds(j * 16, 16)] = fcv
            return 0

        lax.fori_loop(0, P // 16, col_fn, 0)
        pltpu.sync_copy(orow, fci_hbm.at[row])
        pltpu.sync_copy(wrow, w_hbm.at[row])
        pltpu.sync_copy(fcrow, fc_hbm.at[row])
        return 0

    lax.fori_loop(0, _ROWS_PER_W, row_fn, 0)


def _sc_gather(state_flat):
    mesh = plsc.VectorSubcoreMesh(core_axis_name="c", subcore_axis_name="s")
    fn = pl.kernel(
        _sc_gather_body,
        mesh=mesh,
        out_type=[
            jax.ShapeDtypeStruct((B, P), jnp.float32),
            jax.ShapeDtypeStruct((B, P), jnp.float32),
            jax.ShapeDtypeStruct((B, P), jnp.float32),
        ],
        scratch_types=[
            pltpu.VMEM((_RW,), jnp.float32),
            pltpu.VMEM((P,), jnp.float32),
            pltpu.VMEM((P,), jnp.float32),
            pltpu.VMEM((P,), jnp.float32),
        ],
        compiler_params=pltpu.CompilerParams(needs_layout_passes=False),
    )
    return fn(state_flat)


_PK = 8
_RT = 2048
_MP = M // _PK


def _item_head_body(s_ref, f_ref, w1_ref, b1_ref, w2_ref, b2_ref, out_ref):
    s48 = s_ref[...]
    f8 = f_ref[...]
    pieces = []
    for j in range(_PK):
        pieces.append(s48[:, 6 * j + 1:6 * j + 2])
        pieces.append(f8[:, j:j + 1])
        pieces.append(s48[:, 6 * j + 3:6 * j + 6])
    s40 = jnp.concatenate(pieces, axis=1)
    G = lax.dot_general(s40, w1_ref[...], DN,
                        preferred_element_type=jnp.float32)
    h = jnp.maximum(G + b1_ref[...], 0.0)
    l = lax.dot_general(h, w2_ref[...], DN,
                        preferred_element_type=jnp.float32)
    out_ref[...] = l + b2_ref[...]


def _bin_head_body(s_ref, iw_ref, w1_ref, b1_ref, w2_ref, b2_ref, out_ref):
    s48 = s_ref[...]
    rid = lax.broadcasted_iota(jnp.int32, (_RT, 1), 0) // (_RT // _PK)
    iwc = jnp.full((_RT, 1), iw_ref[0, 0])
    for j in range(1, _PK):
        iwc = jnp.where(rid == j, iw_ref[j, 0], iwc)
    pieces = []
    for j in range(_PK):
        pieces.append(iwc)
        pieces.append(s48[:, 6 * j + 2:6 * j + 6])
    s40 = jnp.concatenate(pieces, axis=1)
    G = lax.dot_general(s40, w1_ref[...], DN,
                        preferred_element_type=jnp.float32)
    h = jnp.maximum(G + b1_ref[...], 0.0)
    l = lax.dot_general(h, w2_ref[...], DN,
                        preferred_element_type=jnp.float32)
    out_ref[...] = l + b2_ref[...]


def _packed_weights(W1, b1, W2):
    eye = jnp.eye(_PK, dtype=jnp.float32)
    return (jnp.kron(eye, W1), jnp.tile(b1.reshape(1, D), (1, _PK)),
            jnp.kron(eye, W2))


def _mlp_head_item(state48, fci8, W1, b1, W2, b2):
    w1p, b1p, w2p = _packed_weights(W1, b1, W2)
    out = pl.pallas_call(
        _item_head_body,
        grid=(_MP // _RT,),
        in_specs=[
            pl.BlockSpec((_RT, 48), lambda i: (i, 0)),
            pl.BlockSpec((_RT, _PK), lambda i: (i, 0)),
            pl.BlockSpec((5 * _PK, D * _PK), lambda i: (0, 0)),
            pl.BlockSpec((1, D * _PK), lambda i: (0, 0)),
            pl.BlockSpec((D * _PK, _PK), lambda i: (0, 0)),
            pl.BlockSpec((1, 1), lambda i: (0, 0)),
        ],
        out_specs=pl.BlockSpec((_RT, _PK), lambda i: (i, 0)),
        out_shape=jax.ShapeDtypeStruct((_MP, _PK), jnp.float32),
        compiler_params=pltpu.CompilerParams(
            dimension_semantics=("arbitrary",),
        ),
    )(state48, fci8, w1p, b1p, w2p, b2.reshape(1, 1))
    return out.reshape(B, P)


def _mlp_head_bin(state48, iw, W1, b1, W2, b2):
    w1p, b1p, w2p = _packed_weights(W1, b1, W2)
    out = pl.pallas_call(
        _bin_head_body,
        grid=(_MP // _RT,),
        in_specs=[
            pl.BlockSpec((_RT, 48), lambda i: (i, 0)),
            pl.BlockSpec((_PK, 1), lambda i: (i, 0)),
            pl.BlockSpec((5 * _PK, D * _PK), lambda i: (0, 0)),
            pl.BlockSpec((1, D * _PK), lambda i: (0, 0)),
            pl.BlockSpec((D * _PK, _PK), lambda i: (0, 0)),
            pl.BlockSpec((1, 1), lambda i: (0, 0)),
        ],
        out_specs=pl.BlockSpec((_RT, _PK), lambda i: (i, 0)),
        out_shape=jax.ShapeDtypeStruct((_MP, _PK), jnp.float32),
        compiler_params=pltpu.CompilerParams(
            dimension_semantics=("arbitrary",),
        ),
    )(state48, iw, w1p, b1p, w2p, b2.reshape(1, 1))
    return out.reshape(B, P)


_BT = 128


def _item_sel_body(l_ref, w_ref, item_ref, lp_ref, iw_ref):
    l = l_ref[...]
    m = jnp.max(l, axis=-1, keepdims=True)
    ex = jnp.exp(l - m)
    s = jnp.sum(ex, axis=-1, keepdims=True)
    probs = ex / s
    pm = jnp.max(probs, axis=-1, keepdims=True)
    iota = lax.broadcasted_iota(jnp.int32, l.shape, 1)
    item = jnp.min(jnp.where(probs == pm, iota, P), axis=-1, keepdims=True)
    item_ref[...] = item
    lp_ref[...] = jnp.log(pm)
    iw_ref[...] = jnp.sum(jnp.where(iota == item, w_ref[...], 0.0),
                          axis=-1, keepdims=True)


def _item_select(l_item, wplane):
    return pl.pallas_call(
        _item_sel_body,
        grid=(B // _BT,),
        in_specs=[
            pl.BlockSpec((_BT, P), lambda i: (i, 0)),
            pl.BlockSpec((_BT, P), lambda i: (i, 0)),
        ],
        out_specs=[
            pl.BlockSpec((_BT, 1), lambda i: (i, 0)),
            pl.BlockSpec((_BT, 1), lambda i: (i, 0)),
            pl.BlockSpec((_BT, 1), lambda i: (i, 0)),
        ],
        out_shape=[
            jax.ShapeDtypeStruct((B, 1), jnp.int32),
            jax.ShapeDtypeStruct((B, 1), jnp.float32),
            jax.ShapeDtypeStruct((B, 1), jnp.float32),
        ],
        compiler_params=pltpu.CompilerParams(
            dimension_semantics=("arbitrary",),
        ),
    )(l_item, wplane)


def _bin_sel_body(l_ref, fc_ref, item_ref, iw_ref, lpi_ref, act_ref, lp_ref):
    l = l_ref[...]
    fc = fc_ref[...]
    iw = iw_ref[...]
    item = item_ref[...]
    iota = lax.broadcasted_iota(jnp.int32, l.shape, 1)
    over = jnp.where(iw - fc > 0.0, FMIN, 0.0)
    mask = jnp.where(iota == item, FMIN, over)
    lm = l + mask
    m = jnp.max(lm, axis=-1, keepdims=True)
    ex = jnp.exp(lm - m)
    s = jnp.sum(ex, axis=-1, keepdims=True)
    probs = ex / s
    pm = jnp.max(probs, axis=-1, keepdims=True)
    bin_ = jnp.min(jnp.where(probs == pm, iota, P), axis=-1, keepdims=True)
    act_ref[...] = jnp.concatenate([item, bin_], axis=1)
    lp_ref[...] = lpi_ref[...] + jnp.log(pm)


def _bin_select(l_bin, fcplane, item, iw, lp_i):
    return pl.pallas_call(
        _bin_sel_body,
        grid=(B // _BT,),
        in_specs=[
            pl.BlockSpec((_BT, P), lambda i: (i, 0)),
            pl.BlockSpec((_BT, P), lambda i: (i, 0)),
            pl.BlockSpec((_BT, 1), lambda i: (i, 0)),
            pl.BlockSpec((_BT, 1), lambda i: (i, 0)),
            pl.BlockSpec((_BT, 1), lambda i: (i, 0)),
        ],
        out_specs=[
            pl.BlockSpec((_BT, 2), lambda i: (i, 0)),
            pl.BlockSpec((_BT, 1), lambda i: (i, 0)),
        ],
        out_shape=[
            jax.ShapeDtypeStruct((B, 2), jnp.int32),
            jax.ShapeDtypeStruct((B, 1), jnp.float32),
        ],
        compiler_params=pltpu.CompilerParams(
            dimension_semantics=("arbitrary",),
        ),
    )(l_bin, fcplane, item, iw, lp_i)


def kernel(state, W1i, b1i, W2i, b2i, W1b, b1b, W2b, b2b, greedy=1):
    state_flat = state.reshape(B, P * 6)
    fci, wpl, fcpl = _sc_gather(state_flat)
    state48 = state.reshape(_MP, 48)
    l_item = _mlp_head_item(state48, fci.reshape(_MP, _PK),
                            W1i, b1i, W2i, b2i)
    item, lp_i, iw = _item_select(l_item, wpl)
    l_bin = _mlp_head_bin(state48, iw, W1b, b1b, W2b, b2b)
    action, lp = _bin_select(l_bin, fcpl, item, iw, lp_i)
    return action, lp
